# baseline (speedup 1.0000x reference)
import jax
import jax.numpy as jnp
from jax.experimental import pallas as pl
from jax.experimental.pallas import tpu as pltpu

_B, _S, _H = 4, 4096, 1024
_HID = 512
_L = 119
_LP = 128
_E = 128
_BLK = 2048
_NREP = _S // _BLK


def _mlp_embed_broadcast(last_ref, w1_ref, b1_ref, w2_ref, b2_ref, tab_ref,
                         out_ref, tile_ref, sem):
    x = last_ref[...]
    h = jnp.dot(x, w1_ref[...], preferred_element_type=jnp.float32,
                precision=jax.lax.Precision.HIGHEST)
    h = h + b1_ref[...]
    h = h * 0.5 * (1.0 + jax.lax.erf(h * 0.7071067811865476))
    logits = jnp.dot(h, w2_ref[...], preferred_element_type=jnp.float32,
                     precision=jax.lax.Precision.HIGHEST)
    logits = logits + b2_ref[...]
    m = jnp.max(logits, axis=-1, keepdims=True)
    iota = jax.lax.broadcasted_iota(jnp.int32, logits.shape, 1)
    cand = jnp.where(logits == m, iota, _LP)
    idx = jnp.min(cand, axis=-1, keepdims=True)
    onehot = (iota == idx).astype(jnp.float32)
    emb = jnp.dot(onehot, tab_ref[...],
                  preferred_element_type=jnp.float32,
                  precision=jax.lax.Precision.HIGHEST)

    tile_ref[...] = jnp.broadcast_to(emb[:, None, :], (_B, _BLK, _E))
    copies = [
        pltpu.make_async_copy(
            tile_ref, out_ref.at[:, pl.ds(i * _BLK, _BLK), :], sem)
        for i in range(_NREP)
    ]
    for c in copies:
        c.start()
    for c in copies:
        c.wait()


def kernel(hidden_states, emb_table, W1, b1, W2, b2):
    last = hidden_states[:, -1, :]
    W2p = jnp.pad(W2, ((0, 0), (0, _LP - _L)))
    b2p = jnp.pad(b2, (0, _LP - _L), constant_values=-1e30).reshape(1, _LP)
    tabp = jnp.pad(emb_table, ((0, _LP - _L), (0, 0)))

    out = pl.pallas_call(
        _mlp_embed_broadcast,
        in_specs=[
            pl.BlockSpec(memory_space=pltpu.MemorySpace.VMEM),
            pl.BlockSpec(memory_space=pltpu.MemorySpace.VMEM),
            pl.BlockSpec(memory_space=pltpu.MemorySpace.VMEM),
            pl.BlockSpec(memory_space=pltpu.MemorySpace.VMEM),
            pl.BlockSpec(memory_space=pltpu.MemorySpace.VMEM),
            pl.BlockSpec(memory_space=pltpu.MemorySpace.VMEM),
        ],
        out_specs=pl.BlockSpec(memory_space=pl.ANY),
        out_shape=jax.ShapeDtypeStruct((_B, _S, _E), jnp.float32),
        scratch_shapes=[
            pltpu.VMEM((_B, _BLK, _E), jnp.float32),
            pltpu.SemaphoreType.DMA,
        ],
    )(last, W1, b1.reshape(1, _HID), W2p, b2p, tabp)
    return out


# speedup vs baseline: 1.4825x; 1.4825x over previous
import jax
import jax.numpy as jnp
from jax.experimental import pallas as pl
from jax.experimental.pallas import tpu as pltpu

_B, _S, _H = 4, 4096, 1024
_HID = 512
_L = 119
_E = 128
_BLK = 2048
_NREP = _S // _BLK


def _mlp_embed_broadcast(hs_ref, tab_ref, w1_ref, b1_ref, w2_ref, b2_ref,
                         out_ref, tile_ref, sem):
    x = hs_ref[:, 7, :]
    h = jnp.dot(x, w1_ref[...], preferred_element_type=jnp.float32,
                precision=jax.lax.Precision.HIGHEST)
    h = h + b1_ref[...]
    h = h * 0.5 * (1.0 + jax.lax.erf(h * 0.7071067811865476))
    logits = jnp.dot(h, w2_ref[...], preferred_element_type=jnp.float32,
                     precision=jax.lax.Precision.HIGHEST)
    logits = logits + b2_ref[...]
    m = jnp.max(logits, axis=-1, keepdims=True)
    iota = jax.lax.broadcasted_iota(jnp.int32, logits.shape, 1)
    cand = jnp.where(logits == m, iota, _L)
    idx = jnp.min(cand, axis=-1, keepdims=True)
    onehot = (iota == idx).astype(jnp.float32)
    emb = jnp.dot(onehot, tab_ref[...],
                  preferred_element_type=jnp.float32,
                  precision=jax.lax.Precision.HIGHEST)

    tile_ref[...] = jnp.broadcast_to(emb[:, None, :], (_B, _BLK, _E))
    copies = [
        pltpu.make_async_copy(
            tile_ref, out_ref.at[:, pl.ds(i * _BLK, _BLK), :], sem)
        for i in range(_NREP)
    ]
    for c in copies:
        c.start()
    for c in copies:
        c.wait()


def kernel(hidden_states, emb_table, W1, b1, W2, b2):
    out = pl.pallas_call(
        _mlp_embed_broadcast,
        grid=(1,),
        in_specs=[
            pl.BlockSpec((_B, 8, _H), lambda i: (0, _S // 8 - 1, 0)),
            pl.BlockSpec(memory_space=pltpu.MemorySpace.VMEM),
            pl.BlockSpec(memory_space=pltpu.MemorySpace.VMEM),
            pl.BlockSpec(memory_space=pltpu.MemorySpace.VMEM),
            pl.BlockSpec(memory_space=pltpu.MemorySpace.VMEM),
            pl.BlockSpec(memory_space=pltpu.MemorySpace.VMEM),
        ],
        out_specs=pl.BlockSpec(memory_space=pl.ANY),
        out_shape=jax.ShapeDtypeStruct((_B, _S, _E), jnp.float32),
        scratch_shapes=[
            pltpu.VMEM((_B, _BLK, _E), jnp.float32),
            pltpu.SemaphoreType.DMA,
        ],
    )(hidden_states, emb_table, W1, b1.reshape(1, _HID), W2,
      b2.reshape(1, _L))
    return out


# speedup vs baseline: 1.5260x; 1.0293x over previous
import jax
import jax.numpy as jnp
from jax.experimental import pallas as pl
from jax.experimental.pallas import tpu as pltpu

_B, _S, _H = 4, 4096, 1024
_HID = 512
_L = 119
_E = 128
_BLK = 512
_NREP = _S // _BLK


def _mlp_embed_broadcast(hs_ref, tab_ref, w1_ref, b1_ref, w2_ref, b2_ref,
                         out_ref, tile_ref, sem):
    x = hs_ref[:, 7, :]
    h = jnp.dot(x, w1_ref[...], preferred_element_type=jnp.float32,
                precision=jax.lax.Precision.HIGHEST)
    h = h + b1_ref[...]
    h = h * 0.5 * (1.0 + jax.lax.erf(h * 0.7071067811865476))
    logits = jnp.dot(h, w2_ref[...], preferred_element_type=jnp.float32,
                     precision=jax.lax.Precision.HIGHEST)
    logits = logits + b2_ref[...]
    m = jnp.max(logits, axis=-1, keepdims=True)
    iota = jax.lax.broadcasted_iota(jnp.int32, logits.shape, 1)
    cand = jnp.where(logits == m, iota, _L)
    idx = jnp.min(cand, axis=-1, keepdims=True)
    onehot = (iota == idx).astype(jnp.float32)
    emb = jnp.dot(onehot, tab_ref[...],
                  preferred_element_type=jnp.float32,
                  precision=jax.lax.Precision.HIGHEST)

    tile_ref[...] = jnp.broadcast_to(emb[:, None, :], (_B, _BLK, _E))
    copies = [
        pltpu.make_async_copy(
            tile_ref, out_ref.at[:, pl.ds(i * _BLK, _BLK), :], sem)
        for i in range(_NREP)
    ]
    for c in copies:
        c.start()
    for c in copies:
        c.wait()


def kernel(hidden_states, emb_table, W1, b1, W2, b2):
    out = pl.pallas_call(
        _mlp_embed_broadcast,
        grid=(1,),
        in_specs=[
            pl.BlockSpec((_B, 8, _H), lambda i: (0, _S // 8 - 1, 0)),
            pl.BlockSpec(memory_space=pltpu.MemorySpace.VMEM),
            pl.BlockSpec(memory_space=pltpu.MemorySpace.VMEM),
            pl.BlockSpec(memory_space=pltpu.MemorySpace.VMEM),
            pl.BlockSpec(memory_space=pltpu.MemorySpace.VMEM),
            pl.BlockSpec(memory_space=pltpu.MemorySpace.VMEM),
        ],
        out_specs=pl.BlockSpec(memory_space=pl.ANY),
        out_shape=jax.ShapeDtypeStruct((_B, _S, _E), jnp.float32),
        scratch_shapes=[
            pltpu.VMEM((_B, _BLK, _E), jnp.float32),
            pltpu.SemaphoreType.DMA,
        ],
    )(hidden_states, emb_table, W1, b1.reshape(1, _HID), W2,
      b2.reshape(1, _L))
    return out
